# Initial kernel scaffold; baseline (speedup 1.0000x reference)
import jax
import jax.numpy as jnp
from jax.experimental import pallas as pl


def kernel(x, W):
    raise NotImplementedError("write your pallas kernel here")


# speedup vs baseline: 2.4750x; 2.4750x over previous
import functools

import jax
import jax.numpy as jnp
from jax.experimental import pallas as pl

K = 1024
D = 64
TL = 512


def _vq_block(x_ref, w_ref, q_ref, i_ref):
    xb = x_ref[0]
    w = w_ref[...]
    scores = jax.lax.dot_general(
        w, xb, (((1,), (0,)), ((), ())),
        preferred_element_type=jnp.float32)
    wsq = jnp.sum(w * w, axis=1)
    xsq = jnp.sum(xb * xb, axis=0)
    d2 = (xsq[None, :] + wsq[:, None]) - 2.0 * scores
    minv = jnp.min(d2, axis=0)
    kiota = jax.lax.broadcasted_iota(jnp.int32, (K, TL), 0)
    idx = jnp.min(jnp.where(d2 == minv[None, :], kiota, K), axis=0)
    oneh = (kiota == idx[None, :]).astype(jnp.float32)
    q = jax.lax.dot_general(
        w, oneh, (((0,), (0,)), ((), ())),
        preferred_element_type=jnp.float32)
    q_ref[0] = q
    i_ref[0, 0] = idx


@jax.jit
def kernel(x, W):
    B, Dd, L = x.shape
    nl = L // TL
    grid = (B, nl)
    q, idx = pl.pallas_call(
        _vq_block,
        grid=grid,
        in_specs=[
            pl.BlockSpec((1, Dd, TL), lambda b, l: (b, 0, l)),
            pl.BlockSpec((K, Dd), lambda b, l: (0, 0)),
        ],
        out_specs=[
            pl.BlockSpec((1, Dd, TL), lambda b, l: (b, 0, l)),
            pl.BlockSpec((1, 1, TL), lambda b, l: (b * nl + l, 0, 0)),
        ],
        out_shape=[
            jax.ShapeDtypeStruct((B, Dd, L), jnp.float32),
            jax.ShapeDtypeStruct((B * nl, 1, TL), jnp.int32),
        ],
    )(x, W)
    return q, idx.reshape(B, L)


# speedup vs baseline: 4.3925x; 1.7747x over previous
import functools

import jax
import jax.numpy as jnp
from jax.experimental import pallas as pl

K = 1024
D = 64
TL = 1024


def _vq_block(x_ref, w_ref, q_ref, i_ref):
    xb = x_ref[0]
    w = w_ref[...]
    scores = jax.lax.dot_general(
        w, xb, (((1,), (0,)), ((), ())),
        preferred_element_type=jnp.float32)
    wsq = jnp.sum(w * w, axis=1)
    t = 0.5 * wsq[:, None] - scores
    idx = jnp.argmin(t, axis=0)
    kiota = jax.lax.broadcasted_iota(jnp.int32, (K, TL), 0)
    oneh = (kiota == idx[None, :]).astype(jnp.float32)
    q = jax.lax.dot_general(
        w, oneh, (((0,), (0,)), ((), ())),
        preferred_element_type=jnp.float32)
    q_ref[0] = q
    i_ref[0, 0] = idx


@jax.jit
def kernel(x, W):
    B, Dd, L = x.shape
    nl = L // TL
    grid = (B, nl)
    q, idx = pl.pallas_call(
        _vq_block,
        grid=grid,
        in_specs=[
            pl.BlockSpec((1, Dd, TL), lambda b, l: (b, 0, l)),
            pl.BlockSpec((K, Dd), lambda b, l: (0, 0)),
        ],
        out_specs=[
            pl.BlockSpec((1, Dd, TL), lambda b, l: (b, 0, l)),
            pl.BlockSpec((1, 1, TL), lambda b, l: (b * nl + l, 0, 0)),
        ],
        out_shape=[
            jax.ShapeDtypeStruct((B, Dd, L), jnp.float32),
            jax.ShapeDtypeStruct((B * nl, 1, TL), jnp.int32),
        ],
    )(x, W)
    return q, idx.reshape(B, L)
